# speedup vs baseline: 3.4100x; 3.4100x over previous
import functools

import jax
import jax.numpy as jnp
from jax import lax
from jax.experimental import pallas as pl
from jax.experimental.pallas import tpu as pltpu
from jax.experimental.pallas import tpu_sc as plsc

NC = 2
NS = 16
LANES = 16
NW = NC * NS

B = 4096
EMB = 128
LCODES = 50
MED = 1000

BPW = B // NW
G = 2
NG = BPW // G
GI = G * LCODES


def _emb_body(dc_hbm, pc_hbm, dtab_hbm, ptab_hbm, out_hbm,
              idx_d, idx_p, rows, acc, sem):
    wid = lax.axis_index("s") * NC + lax.axis_index("c")
    pltpu.sync_copy(dc_hbm.at[wid], idx_d)
    pltpu.sync_copy(pc_hbm.at[wid], idx_p)

    def do_table(idx_v, tab_hbm, c0):
        def group(g, carry):
            pltpu.async_copy(tab_hbm.at[idx_v.at[g]], rows, sem).wait()
            for v in range(G):
                for c in range(EMB // LANES):
                    accv = rows[v * LCODES, pl.ds(c * LANES, LANES)]
                    for i in range(1, LCODES):
                        accv = accv + rows[v * LCODES + i,
                                           pl.ds(c * LANES, LANES)]
                    acc[g * G + v, pl.ds(c0 + c * LANES, LANES)] = accv
            return carry
        lax.fori_loop(0, NG, group, 0)

    do_table(idx_d, dtab_hbm, 0)
    do_table(idx_p, ptab_hbm, EMB)
    pltpu.sync_copy(acc, out_hbm.at[wid])


_emb = pl.kernel(
    _emb_body,
    out_type=jax.ShapeDtypeStruct((NW, BPW, 2 * EMB), jnp.float32),
    mesh=plsc.VectorSubcoreMesh(
        core_axis_name="c", subcore_axis_name="s",
        num_cores=NC, num_subcores=NS),
    scratch_types=[
        pltpu.VMEM((NG, GI), jnp.int32),
        pltpu.VMEM((NG, GI), jnp.int32),
        pltpu.VMEM((GI, EMB), jnp.float32),
        pltpu.VMEM((BPW, 2 * EMB), jnp.float32),
        pltpu.SemaphoreType.DMA,
    ],
)


def _mlp_body(comb_ref, w1t_ref, b1_ref, w2t_ref, b2_ref, out_ref):
    x = comb_ref[...]
    h = jnp.dot(x, w1t_ref[...], preferred_element_type=jnp.float32)
    h = jnp.maximum(h + b1_ref[...], 0.0)
    z = jnp.dot(h, w2t_ref[...], preferred_element_type=jnp.float32)
    z = z + b2_ref[...]
    out_ref[...] = 1.0 / (1.0 + jnp.exp(-z))


_ROWS_BLK = 1024

_mlp = pl.pallas_call(
    _mlp_body,
    out_shape=jax.ShapeDtypeStruct((B, MED), jnp.float32),
    grid=(B // _ROWS_BLK,),
    in_specs=[
        pl.BlockSpec((_ROWS_BLK, 2 * EMB), lambda i: (i, 0)),
        pl.BlockSpec((2 * EMB, 64), lambda i: (0, 0)),
        pl.BlockSpec((1, 64), lambda i: (0, 0)),
        pl.BlockSpec((64, MED), lambda i: (0, 0)),
        pl.BlockSpec((1, MED), lambda i: (0, 0)),
    ],
    out_specs=pl.BlockSpec((_ROWS_BLK, MED), lambda i: (i, 0)),
)


@jax.jit
def kernel(diag_codes, proc_codes, diag_table, proc_table, W1, b1, W2, b2):
    dc = diag_codes.reshape(NW, NG, GI)
    pc = proc_codes.reshape(NW, NG, GI)
    comb = _emb(dc, pc, diag_table, proc_table).reshape(B, 2 * EMB)
    return _mlp(comb, W1.T, b1.reshape(1, 64), W2.T, b2.reshape(1, MED))


# speedup vs baseline: 4.9956x; 1.4650x over previous
import functools

import jax
import jax.numpy as jnp
from jax import lax
from jax.experimental import pallas as pl
from jax.experimental.pallas import tpu as pltpu
from jax.experimental.pallas import tpu_sc as plsc

NC = 2
NS = 16
LANES = 16
NW = NC * NS

B = 4096
EMB = 128
LCODES = 50
MED = 1000

BPW = B // NW
G = 2
NG = BPW // G
GI = G * LCODES


NBUF = 4


def _emb_body(dc_hbm, pc_hbm, dtab_hbm, ptab_hbm, out_hbm,
              idx_d, idx_p, rows, acc, sem):
    wid = lax.axis_index("s") * NC + lax.axis_index("c")
    pltpu.sync_copy(dc_hbm.at[wid], idx_d)
    pltpu.sync_copy(pc_hbm.at[wid], idx_p)

    def do_table(idx_v, tab_hbm, c0):
        for p in range(NBUF - 1):
            pltpu.async_copy(tab_hbm.at[idx_v.at[p]], rows.at[p], sem)

        def group(g, carry):
            b = lax.rem(g, NBUF)
            pltpu.make_async_copy(
                tab_hbm.at[idx_v.at[g]], rows.at[b], sem).wait()
            nxt = g + (NBUF - 1)

            @pl.when(nxt < NG)
            def _():
                pltpu.async_copy(
                    tab_hbm.at[idx_v.at[nxt]],
                    rows.at[lax.rem(nxt, NBUF)], sem)

            for v in range(G):
                for c in range(EMB // LANES):
                    accv = rows[b, v * LCODES, pl.ds(c * LANES, LANES)]
                    for i in range(1, LCODES):
                        accv = accv + rows[b, v * LCODES + i,
                                           pl.ds(c * LANES, LANES)]
                    acc[g * G + v, pl.ds(c0 + c * LANES, LANES)] = accv
            return carry
        lax.fori_loop(0, NG, group, 0)

    do_table(idx_d, dtab_hbm, 0)
    do_table(idx_p, ptab_hbm, EMB)
    pltpu.sync_copy(acc, out_hbm.at[wid])


_emb = pl.kernel(
    _emb_body,
    out_type=jax.ShapeDtypeStruct((NW, BPW, 2 * EMB), jnp.float32),
    mesh=plsc.VectorSubcoreMesh(
        core_axis_name="c", subcore_axis_name="s",
        num_cores=NC, num_subcores=NS),
    scratch_types=[
        pltpu.VMEM((NG, GI), jnp.int32),
        pltpu.VMEM((NG, GI), jnp.int32),
        pltpu.VMEM((NBUF, GI, EMB), jnp.float32),
        pltpu.VMEM((BPW, 2 * EMB), jnp.float32),
        pltpu.SemaphoreType.DMA,
    ],
)


def _mlp_body(comb_ref, w1t_ref, b1_ref, w2t_ref, b2_ref, out_ref):
    x = comb_ref[...]
    h = jnp.dot(x, w1t_ref[...], preferred_element_type=jnp.float32)
    h = jnp.maximum(h + b1_ref[...], 0.0)
    z = jnp.dot(h, w2t_ref[...], preferred_element_type=jnp.float32)
    z = z + b2_ref[...]
    out_ref[...] = 1.0 / (1.0 + jnp.exp(-z))


_ROWS_BLK = 1024

_mlp = pl.pallas_call(
    _mlp_body,
    out_shape=jax.ShapeDtypeStruct((B, MED), jnp.float32),
    grid=(B // _ROWS_BLK,),
    in_specs=[
        pl.BlockSpec((_ROWS_BLK, 2 * EMB), lambda i: (i, 0)),
        pl.BlockSpec((2 * EMB, 64), lambda i: (0, 0)),
        pl.BlockSpec((1, 64), lambda i: (0, 0)),
        pl.BlockSpec((64, MED), lambda i: (0, 0)),
        pl.BlockSpec((1, MED), lambda i: (0, 0)),
    ],
    out_specs=pl.BlockSpec((_ROWS_BLK, MED), lambda i: (i, 0)),
)


@jax.jit
def kernel(diag_codes, proc_codes, diag_table, proc_table, W1, b1, W2, b2):
    dc = diag_codes.reshape(NW, NG, GI)
    pc = proc_codes.reshape(NW, NG, GI)
    comb = _emb(dc, pc, diag_table, proc_table).reshape(B, 2 * EMB)
    return _mlp(comb, W1.T, b1.reshape(1, 64), W2.T, b2.reshape(1, MED))


# speedup vs baseline: 13.7416x; 2.7507x over previous
import functools

import jax
import jax.numpy as jnp
from jax import lax
from jax.experimental import pallas as pl
from jax.experimental.pallas import tpu as pltpu
from jax.experimental.pallas import tpu_sc as plsc

NC = 2
NS = 16
LANES = 16
NW = NC * NS

B = 4096
EMB = 128
LCODES = 50
MED = 1000

BPW = B // NW
G = 2
NG = BPW // G
GI = G * LCODES


NBUF = 4


def _emb_body(dc_hbm, pc_hbm, dtab_hbm, ptab_hbm, out_hbm,
              idx_d, idx_p, rows, acc, sem):
    wid = lax.axis_index("s") * NC + lax.axis_index("c")
    pltpu.sync_copy(dc_hbm.at[wid], idx_d)
    pltpu.sync_copy(pc_hbm.at[wid], idx_p)

    def do_table(idx_v, tab_hbm, c0):
        for p in range(NBUF - 1):
            pltpu.async_copy(tab_hbm.at[idx_v.at[p]], rows.at[p], sem)

        def group(g, carry):
            b = lax.rem(g, NBUF)
            pltpu.make_async_copy(
                tab_hbm.at[idx_v.at[g]], rows.at[b], sem).wait()
            nxt = g + (NBUF - 1)

            @pl.when(nxt < NG)
            def _():
                pltpu.async_copy(
                    tab_hbm.at[idx_v.at[nxt]],
                    rows.at[lax.rem(nxt, NBUF)], sem)

            NCH = EMB // LANES
            HALF = NCH // 4
            for v in range(G):
                for h in range(4):
                    cs = [h * HALF + c for c in range(HALF)]
                    accs = [rows[b, v * LCODES, pl.ds(c * LANES, LANES)]
                            for c in cs]
                    for i in range(1, LCODES):
                        for j, c in enumerate(cs):
                            accs[j] = accs[j] + rows[b, v * LCODES + i,
                                                     pl.ds(c * LANES, LANES)]
                    for j, c in enumerate(cs):
                        acc[g * G + v, pl.ds(c0 + c * LANES, LANES)] = accs[j]
            return carry
        lax.fori_loop(0, NG, group, 0)

    do_table(idx_d, dtab_hbm, 0)
    do_table(idx_p, ptab_hbm, EMB)
    pltpu.sync_copy(acc, out_hbm.at[wid])


_emb = pl.kernel(
    _emb_body,
    out_type=jax.ShapeDtypeStruct((NW, BPW, 2 * EMB), jnp.float32),
    mesh=plsc.VectorSubcoreMesh(
        core_axis_name="c", subcore_axis_name="s",
        num_cores=NC, num_subcores=NS),
    scratch_types=[
        pltpu.VMEM((NG, GI), jnp.int32),
        pltpu.VMEM((NG, GI), jnp.int32),
        pltpu.VMEM((NBUF, GI, EMB), jnp.float32),
        pltpu.VMEM((BPW, 2 * EMB), jnp.float32),
        pltpu.SemaphoreType.DMA,
    ],
)


def _mlp_body(comb_ref, w1t_ref, b1_ref, w2t_ref, b2_ref, out_ref):
    x = comb_ref[...]
    h = jnp.dot(x, w1t_ref[...], preferred_element_type=jnp.float32)
    h = jnp.maximum(h + b1_ref[...], 0.0)
    z = jnp.dot(h, w2t_ref[...], preferred_element_type=jnp.float32)
    z = z + b2_ref[...]
    out_ref[...] = 1.0 / (1.0 + jnp.exp(-z))


_ROWS_BLK = 1024

_mlp = pl.pallas_call(
    _mlp_body,
    out_shape=jax.ShapeDtypeStruct((B, MED), jnp.float32),
    grid=(B // _ROWS_BLK,),
    in_specs=[
        pl.BlockSpec((_ROWS_BLK, 2 * EMB), lambda i: (i, 0)),
        pl.BlockSpec((2 * EMB, 64), lambda i: (0, 0)),
        pl.BlockSpec((1, 64), lambda i: (0, 0)),
        pl.BlockSpec((64, MED), lambda i: (0, 0)),
        pl.BlockSpec((1, MED), lambda i: (0, 0)),
    ],
    out_specs=pl.BlockSpec((_ROWS_BLK, MED), lambda i: (i, 0)),
)


@jax.jit
def kernel(diag_codes, proc_codes, diag_table, proc_table, W1, b1, W2, b2):
    dc = diag_codes.reshape(NW, NG, GI)
    pc = proc_codes.reshape(NW, NG, GI)
    comb = _emb(dc, pc, diag_table, proc_table).reshape(B, 2 * EMB)
    return _mlp(comb, W1.T, b1.reshape(1, 64), W2.T, b2.reshape(1, MED))


# speedup vs baseline: 13.7891x; 1.0035x over previous
import functools

import jax
import jax.numpy as jnp
from jax import lax
from jax.experimental import pallas as pl
from jax.experimental.pallas import tpu as pltpu
from jax.experimental.pallas import tpu_sc as plsc

NC = 2
NS = 16
LANES = 16
NW = NC * NS

B = 4096
EMB = 128
LCODES = 50
MED = 1000

BPW = B // NW
G = 2
NG = BPW // G
GI = G * LCODES


NBUF = 6


def _emb_body(dc_hbm, pc_hbm, dtab_hbm, ptab_hbm, out_hbm,
              idx_d, idx_p, rows, acc, sem):
    wid = lax.axis_index("s") * NC + lax.axis_index("c")
    pltpu.sync_copy(dc_hbm.at[wid], idx_d)
    pltpu.sync_copy(pc_hbm.at[wid], idx_p)

    def do_table(idx_v, tab_hbm, c0):
        for p in range(NBUF - 1):
            pltpu.async_copy(tab_hbm.at[idx_v.at[p]], rows.at[p], sem)

        def group(g, carry):
            b = lax.rem(g, NBUF)
            pltpu.make_async_copy(
                tab_hbm.at[idx_v.at[g]], rows.at[b], sem).wait()
            nxt = g + (NBUF - 1)

            @pl.when(nxt < NG)
            def _():
                pltpu.async_copy(
                    tab_hbm.at[idx_v.at[nxt]],
                    rows.at[lax.rem(nxt, NBUF)], sem)

            NCH = EMB // LANES
            HALF = NCH // 4
            for v in range(G):
                for h in range(4):
                    cs = [h * HALF + c for c in range(HALF)]
                    accs = [rows[b, v * LCODES, pl.ds(c * LANES, LANES)]
                            for c in cs]
                    for i in range(1, LCODES):
                        for j, c in enumerate(cs):
                            accs[j] = accs[j] + rows[b, v * LCODES + i,
                                                     pl.ds(c * LANES, LANES)]
                    for j, c in enumerate(cs):
                        acc[g * G + v, pl.ds(c0 + c * LANES, LANES)] = accs[j]
            return carry
        lax.fori_loop(0, NG, group, 0)

    do_table(idx_d, dtab_hbm, 0)
    do_table(idx_p, ptab_hbm, EMB)
    pltpu.sync_copy(acc, out_hbm.at[wid])


_emb = pl.kernel(
    _emb_body,
    out_type=jax.ShapeDtypeStruct((NW, BPW, 2 * EMB), jnp.float32),
    mesh=plsc.VectorSubcoreMesh(
        core_axis_name="c", subcore_axis_name="s",
        num_cores=NC, num_subcores=NS),
    scratch_types=[
        pltpu.VMEM((NG, GI), jnp.int32),
        pltpu.VMEM((NG, GI), jnp.int32),
        pltpu.VMEM((NBUF, GI, EMB), jnp.float32),
        pltpu.VMEM((BPW, 2 * EMB), jnp.float32),
        pltpu.SemaphoreType.DMA,
    ],
)


def _mlp_body(comb_ref, w1t_ref, b1_ref, w2t_ref, b2_ref, out_ref):
    x = comb_ref[...]
    h = jnp.dot(x, w1t_ref[...], preferred_element_type=jnp.float32)
    h = jnp.maximum(h + b1_ref[...], 0.0)
    z = jnp.dot(h, w2t_ref[...], preferred_element_type=jnp.float32)
    z = z + b2_ref[...]
    out_ref[...] = 1.0 / (1.0 + jnp.exp(-z))


_ROWS_BLK = 1024

_mlp = pl.pallas_call(
    _mlp_body,
    out_shape=jax.ShapeDtypeStruct((B, MED), jnp.float32),
    grid=(B // _ROWS_BLK,),
    in_specs=[
        pl.BlockSpec((_ROWS_BLK, 2 * EMB), lambda i: (i, 0)),
        pl.BlockSpec((2 * EMB, 64), lambda i: (0, 0)),
        pl.BlockSpec((1, 64), lambda i: (0, 0)),
        pl.BlockSpec((64, MED), lambda i: (0, 0)),
        pl.BlockSpec((1, MED), lambda i: (0, 0)),
    ],
    out_specs=pl.BlockSpec((_ROWS_BLK, MED), lambda i: (i, 0)),
)


@jax.jit
def kernel(diag_codes, proc_codes, diag_table, proc_table, W1, b1, W2, b2):
    dc = diag_codes.reshape(NW, NG, GI)
    pc = proc_codes.reshape(NW, NG, GI)
    comb = _emb(dc, pc, diag_table, proc_table).reshape(B, 2 * EMB)
    return _mlp(comb, W1.T, b1.reshape(1, 64), W2.T, b2.reshape(1, MED))
